# baseline (speedup 1.0000x reference)
import functools

import jax
import jax.numpy as jnp
from jax import lax
from jax.experimental import pallas as pl
from jax.experimental.pallas import tpu as pltpu
from jax.experimental.pallas import tpu_sc as plsc

NC = 2
NS = 16
L = 16
NW = NC * NS
BATCH = 16384
D = 16
BPW = BATCH // NW
CHUNK = 128
NCHUNK = BPW // CHUNK
GROUPS = BPW // L


def _bprmf_body(u_hbm, i_hbm, ue_hbm, ie_hbm, out_hbm,
                u_idx, i_idx, u_rows, i_rows, out_v, sem):
    wid = lax.axis_index("s") * NC + lax.axis_index("c")

    pltpu.sync_copy(u_hbm.at[pl.ds(wid * NCHUNK, NCHUNK)], u_idx)
    pltpu.sync_copy(i_hbm.at[pl.ds(wid * NCHUNK, NCHUNK)], i_idx)

    copies = []
    for j in range(NCHUNK):
        copies.append(pltpu.async_copy(
            ue_hbm.at[u_idx.at[j]], u_rows.at[pl.ds(j * CHUNK, CHUNK)], sem))
        copies.append(pltpu.async_copy(
            ie_hbm.at[i_idx.at[j]], i_rows.at[pl.ds(j * CHUNK, CHUNK)], sem))
    for c in copies:
        c.wait()

    lanes = lax.iota(jnp.int32, L)

    def group(g, _):
        row_ids = g * L + lanes
        acc = jnp.zeros((L,), jnp.float32)
        for d in range(D):
            dvec = jnp.full((L,), d, jnp.int32)
            gu = plsc.load_gather(u_rows, [row_ids, dvec])
            gi = plsc.load_gather(i_rows, [row_ids, dvec])
            acc = acc + gu * gi
        out_v[pl.ds(g * L, L)] = acc
        return _

    lax.fori_loop(0, GROUPS, group, None)

    pltpu.sync_copy(out_v, out_hbm.at[pl.ds(wid * BPW, BPW)])


@jax.jit
def kernel(u, i, user_emb, item_emb):
    mesh = plsc.VectorSubcoreMesh(core_axis_name="c", subcore_axis_name="s")
    f = pl.kernel(
        _bprmf_body,
        out_type=jax.ShapeDtypeStruct((BATCH,), jnp.float32),
        mesh=mesh,
        compiler_params=pltpu.CompilerParams(
            needs_layout_passes=False, use_tc_tiling_on_sc=False),
        scratch_types=[
            pltpu.VMEM((NCHUNK, CHUNK), jnp.int32),
            pltpu.VMEM((NCHUNK, CHUNK), jnp.int32),
            pltpu.VMEM((BPW, D), jnp.float32),
            pltpu.VMEM((BPW, D), jnp.float32),
            pltpu.VMEM((BPW,), jnp.float32),
            pltpu.SemaphoreType.DMA,
        ],
    )
    u2 = u.reshape(NW * NCHUNK, CHUNK)
    i2 = i.reshape(NW * NCHUNK, CHUNK)
    return f(u2, i2, user_emb, item_emb)
